# baseline (speedup 1.0000x reference)
import jax
import jax.numpy as jnp
from jax.experimental import pallas as pl

_BR = 40


def _mask_body(x_ref, mv_ref, o_ref):
    i = pl.program_id(0)
    x = x_ref[...]
    mv = mv_ref[0, 0, :]
    row = jax.lax.broadcasted_iota(jnp.int32, x.shape, 0) + i * _BR
    o_ref[...] = jnp.where(row == mv[None, :], jnp.float32(-1.0), x)


def kernel(input, masked_values):
    B, C = input.shape
    inp_t = input.T
    grid = (C // _BR,)
    mv3 = masked_values.reshape(1, 1, B)
    out_t = pl.pallas_call(
        _mask_body,
        grid=grid,
        in_specs=[
            pl.BlockSpec((_BR, B), lambda i: (i, 0)),
            pl.BlockSpec((1, 1, B), lambda i: (0, 0, 0)),
        ],
        out_specs=pl.BlockSpec((_BR, B), lambda i: (i, 0)),
        out_shape=jax.ShapeDtypeStruct((C, B), input.dtype),
    )(inp_t, mv3)
    return out_t.T


# speedup vs baseline: 1.1196x; 1.1196x over previous
import jax
import jax.numpy as jnp
from jax.experimental import pallas as pl
from jax.experimental.pallas import tpu as pltpu

_BC = 4096


def _mask_body(x_ref, mv_ref, o_ref):
    x = x_ref[...]
    mv = mv_ref[0, 0, :]
    row = jax.lax.broadcasted_iota(jnp.int32, x.shape, 0)
    o_ref[...] = jnp.where(row == mv[None, :], jnp.float32(-1.0), x)


def kernel(input, masked_values):
    B, C = input.shape
    inp_t = input.T
    grid = (B // _BC,)
    mv3 = masked_values.reshape(grid[0], 1, _BC)
    out_t = pl.pallas_call(
        _mask_body,
        grid=grid,
        compiler_params=pltpu.CompilerParams(vmem_limit_bytes=100 * 1024 * 1024),
        in_specs=[
            pl.BlockSpec((C, _BC), lambda i: (0, i)),
            pl.BlockSpec((1, 1, _BC), lambda i: (i, 0, 0)),
        ],
        out_specs=pl.BlockSpec((C, _BC), lambda i: (0, i)),
        out_shape=jax.ShapeDtypeStruct((C, B), input.dtype),
    )(inp_t, mv3)
    return out_t.T


# speedup vs baseline: 1.1198x; 1.0001x over previous
import jax
import jax.numpy as jnp
from jax.experimental import pallas as pl
from jax.experimental.pallas import tpu as pltpu

_BC = 4096


def _mask_body(x_ref, mv_ref, o_ref):
    x = x_ref[...]
    mv = mv_ref[0, 0, :]
    row = jax.lax.broadcasted_iota(jnp.int32, x.shape, 0)
    o_ref[...] = jnp.where(row == mv[None, :], jnp.float32(-1.0), x)


def kernel(input, masked_values):
    B, C = input.shape
    inp_t = input.T
    grid = (B // _BC,)
    mv3 = masked_values.reshape(grid[0], 1, _BC)
    out_t = pl.pallas_call(
        _mask_body,
        grid=grid,
        compiler_params=pltpu.CompilerParams(vmem_limit_bytes=100 * 1024 * 1024, dimension_semantics=("parallel",)),
        in_specs=[
            pl.BlockSpec((C, _BC), lambda i: (0, i)),
            pl.BlockSpec((1, 1, _BC), lambda i: (i, 0, 0)),
        ],
        out_specs=pl.BlockSpec((C, _BC), lambda i: (0, i)),
        out_shape=jax.ShapeDtypeStruct((C, B), input.dtype),
    )(inp_t, mv3)
    return out_t.T
